# Initial kernel scaffold; baseline (speedup 1.0000x reference)
import jax
import jax.numpy as jnp
from jax.experimental import pallas as pl


def kernel(x, edge_index, bn1_g, bn1_b, Wq, Wk, Wv, Wo, bo, bn2_g, bn2_b, W1, W2):
    raise NotImplementedError("write your pallas kernel here")


# speedup vs baseline: 1.9881x; 1.9881x over previous
import jax
import jax.numpy as jnp
from jax import lax
from jax.experimental import pallas as pl
from jax.experimental.pallas import tpu as pltpu

N = 10000
E = 160000
D = 256
H = 4
EPS_BN = 1e-5
EPS_Z = 1e-6

BLK = 400
GRID = N // BLK
EB = 1000
NEB = E // EB
NB = 400
NNB = N // NB
RW = 80
_PREC = lax.Precision.HIGHEST
_GPREC = lax.Precision.DEFAULT


def _stats_body(x_ref, o_ref):
    i = pl.program_id(0)

    @pl.when(i == 0)
    def _():
        o_ref[...] = jnp.zeros_like(o_ref)

    xb = x_ref[...]
    o_ref[0:1, :] += jnp.sum(xb, axis=0, keepdims=True)
    o_ref[1:2, :] += jnp.sum(xb * xb, axis=0, keepdims=True)


def _stats(x):
    return pl.pallas_call(
        _stats_body,
        grid=(GRID,),
        in_specs=[pl.BlockSpec((BLK, D), lambda i: (i, 0))],
        out_specs=pl.BlockSpec((8, D), lambda i: (0, 0)),
        out_shape=jax.ShapeDtypeStruct((8, D), jnp.float32),
    )(x)


def _qkv_body(x_ref, ab_ref, wq_ref, wk_ref, wv_ref, q_ref, k_ref, v_ref):
    xn = x_ref[...] * ab_ref[0:1, :] + ab_ref[1:2, :]
    q_ref[...] = jnp.dot(xn, wq_ref[...], precision=_PREC)
    k_ref[...] = jnp.dot(xn, wk_ref[...], precision=_PREC)
    v_ref[...] = jnp.dot(xn, wv_ref[...], precision=_PREC)


def _qkv(x, ab1, Wq, Wk, Wv):
    return pl.pallas_call(
        _qkv_body,
        grid=(GRID,),
        in_specs=[
            pl.BlockSpec((BLK, D), lambda i: (i, 0)),
            pl.BlockSpec((8, D), lambda i: (0, 0)),
            pl.BlockSpec((D, D), lambda i: (0, 0)),
            pl.BlockSpec((D, D), lambda i: (0, 0)),
            pl.BlockSpec((D, D), lambda i: (0, 0)),
        ],
        out_specs=[pl.BlockSpec((BLK, D), lambda i: (i, 0))] * 3,
        out_shape=[jax.ShapeDtypeStruct((N, D), jnp.float32)] * 3,
    )(x, ab1, Wq, Wk, Wv)


def _edges_body(src_ref, dst_ref, q_nb, k_nb, v_nb, r_ref, ks, qs, vs):
    j = pl.program_id(1)

    @pl.when(j == 0)
    def _():
        ks[...] = jnp.zeros_like(ks)
        qs[...] = jnp.zeros_like(qs)
        vs[...] = jnp.zeros_like(vs)

    srcv = src_ref[0, 0, :]
    dstv = dst_ref[0, 0, :]
    base = j * NB
    ids = base + lax.broadcasted_iota(jnp.int32, (EB, NB), 1)
    one_s = (srcv[:, None] == ids).astype(jnp.float32)
    one_d = (dstv[:, None] == ids).astype(jnp.float32)
    ks[...] += jnp.dot(one_s, k_nb[...], precision=_GPREC)
    qs[...] += jnp.dot(one_d, q_nb[...], precision=_GPREC)
    vs[...] += jnp.dot(one_s, v_nb[...], precision=_GPREC)

    @pl.when(j == NNB - 1)
    def _():
        pieces = []
        zpad = jnp.zeros((EB, RW - 65), jnp.float32)
        for h in range(H):
            sl = slice(h * 64, (h + 1) * 64)
            dot = jnp.sum(ks[:, sl] * qs[:, sl], axis=1, keepdims=True)
            s = jnp.exp(jnp.clip(dot * 0.125, -5.0, 5.0))
            pieces.extend([vs[:, sl] * s, s, zpad])
        r_ref[...] = jnp.concatenate(pieces, axis=1)


def _edge_rows(src3, dst3, q, k, v):
    return pl.pallas_call(
        _edges_body,
        grid=(NEB, NNB),
        in_specs=[
            pl.BlockSpec((1, 1, EB), lambda i, j: (i, 0, 0)),
            pl.BlockSpec((1, 1, EB), lambda i, j: (i, 0, 0)),
            pl.BlockSpec((NB, D), lambda i, j: (j, 0)),
            pl.BlockSpec((NB, D), lambda i, j: (j, 0)),
            pl.BlockSpec((NB, D), lambda i, j: (j, 0)),
        ],
        out_specs=pl.BlockSpec((EB, H * RW), lambda i, j: (i, 0)),
        out_shape=jax.ShapeDtypeStruct((E, H * RW), jnp.float32),
        scratch_shapes=[pltpu.VMEM((EB, D), jnp.float32)] * 3,
    )(src3, dst3, q, k, v)


def _segsum_body(dst_ref, r_ref, o_ref, acc):
    k = pl.program_id(1)

    @pl.when(k == 0)
    def _():
        acc[...] = jnp.zeros_like(acc)

    i = pl.program_id(0)
    dstv = dst_ref[0, 0, :]
    ids = i * BLK + lax.broadcasted_iota(jnp.int32, (EB, BLK), 1)
    one_d = (dstv[:, None] == ids).astype(jnp.float32)
    acc[...] += lax.dot_general(one_d, r_ref[...],
                                (((0,), (0,)), ((), ())),
                                precision=_GPREC)

    @pl.when(k == NEB - 1)
    def _():
        o_ref[...] = acc[...]


def _segsum(dst3, r):
    return pl.pallas_call(
        _segsum_body,
        grid=(GRID, NEB),
        in_specs=[
            pl.BlockSpec((1, 1, EB), lambda i, k: (k, 0, 0)),
            pl.BlockSpec((EB, H * RW), lambda i, k: (k, 0)),
        ],
        out_specs=pl.BlockSpec((BLK, H * RW), lambda i, k: (i, 0)),
        out_shape=jax.ShapeDtypeStruct((N, H * RW), jnp.float32),
        scratch_shapes=[pltpu.VMEM((BLK, H * RW), jnp.float32)],
    )(dst3, r)


def _comb_body(x_ref, parts_ref, wo_ref, bo_ref, h_ref, st_ref):
    i = pl.program_id(0)

    @pl.when(i == 0)
    def _():
        st_ref[...] = jnp.zeros_like(st_ref)

    pieces = []
    for hh in range(H):
        ph = parts_ref[:, hh * RW:(hh + 1) * RW]
        zc = ph[:, 64:65] + EPS_Z
        pieces.append(ph[:, :64] / zc)
    h2 = jnp.concatenate(pieces, axis=1)
    h = x_ref[...] + jnp.dot(h2, wo_ref[...], precision=_PREC) + bo_ref[0:1, :]
    h_ref[...] = h
    st_ref[0:1, :] += jnp.sum(h, axis=0, keepdims=True)
    st_ref[1:2, :] += jnp.sum(h * h, axis=0, keepdims=True)


def _combine(x, parts, Wo, bo8):
    return pl.pallas_call(
        _comb_body,
        grid=(GRID,),
        in_specs=[
            pl.BlockSpec((BLK, D), lambda i: (i, 0)),
            pl.BlockSpec((BLK, H * RW), lambda i: (i, 0)),
            pl.BlockSpec((D, D), lambda i: (0, 0)),
            pl.BlockSpec((8, D), lambda i: (0, 0)),
        ],
        out_specs=[
            pl.BlockSpec((BLK, D), lambda i: (i, 0)),
            pl.BlockSpec((8, D), lambda i: (0, 0)),
        ],
        out_shape=[
            jax.ShapeDtypeStruct((N, D), jnp.float32),
            jax.ShapeDtypeStruct((8, D), jnp.float32),
        ],
    )(x, parts, Wo, bo8)


def _ffn_body(h_ref, ab_ref, w1_ref, w2_ref, y_ref):
    h = h_ref[...]
    hn = h * ab_ref[0:1, :] + ab_ref[1:2, :]
    u = jnp.dot(hn, w1_ref[...], precision=_PREC)
    s = u * jax.nn.sigmoid(u)
    y_ref[...] = h + jnp.dot(s, w2_ref[...], precision=_PREC)


def _ffn(h, ab2, W1, W2):
    return pl.pallas_call(
        _ffn_body,
        grid=(GRID,),
        in_specs=[
            pl.BlockSpec((BLK, D), lambda i: (i, 0)),
            pl.BlockSpec((8, D), lambda i: (0, 0)),
            pl.BlockSpec((D, 2 * D), lambda i: (0, 0)),
            pl.BlockSpec((2 * D, D), lambda i: (0, 0)),
        ],
        out_specs=pl.BlockSpec((BLK, D), lambda i: (i, 0)),
        out_shape=jax.ShapeDtypeStruct((N, D), jnp.float32),
    )(h, ab2, W1, W2)


def _bn_scale_shift(stats, gamma, beta):
    mean = stats[0] / N
    var = stats[1] / N - mean * mean
    a = gamma * lax.rsqrt(var + EPS_BN)
    b = beta - mean * a
    return jnp.concatenate(
        [a[None], b[None], jnp.zeros((6, D), jnp.float32)], axis=0)


def kernel(x, edge_index, bn1_g, bn1_b, Wq, Wk, Wv, Wo, bo, bn2_g, bn2_b,
           W1, W2):
    ab1 = _bn_scale_shift(_stats(x), bn1_g, bn1_b)
    q, k, v = _qkv(x, ab1, Wq, Wk, Wv)
    src3 = edge_index[0].reshape(NEB, 1, EB)
    dst3 = edge_index[1].reshape(NEB, 1, EB)
    r = _edge_rows(src3, dst3, q, k, v)
    parts = _segsum(dst3, r)
    bo8 = jnp.concatenate([bo[None], jnp.zeros((7, D), jnp.float32)], axis=0)
    h, st2 = _combine(x, parts, Wo, bo8)
    ab2 = _bn_scale_shift(st2, bn2_g, bn2_b)
    return _ffn(h, ab2, W1, W2)
